# baseline (speedup 1.0000x reference)
import functools

import jax
import jax.numpy as jnp
from jax import lax
from jax.experimental import pallas as pl
from jax.experimental.pallas import tpu as pltpu
from jax.experimental.pallas import tpu_sc as plsc

_X, _Y, _Z = 38, 24, 24
_V = _X * _Y * _Z
_VOXEL = (0.3, 0.3, 0.2)
_GROUND = (-5.6, -3.6, -2.4)
_DIMS = (_X, _Y, _Z)

_NC, _NS, _L = 2, 16, 16
_NW = _NC * _NS


def _idx_body(x_ref, y_ref, z_ref, o_ref):
    comps = []
    for ref, vs, g, dim in zip((x_ref, y_ref, z_ref), _VOXEL, _GROUND, _DIMS):
        vsf = jnp.float32(vs)
        mn = jnp.floor(jnp.float32(g) / vsf)
        d = jnp.floor(ref[...] / vsf)
        vi = (d - mn).astype(jnp.int32)
        comps.append(jnp.clip(vi, 0, dim - 1))
    o_ref[...] = comps[0] * (_Y * _Z) + comps[1] * _Z + comps[2]


def _flat_idx(x, y, z):
    B, N = x.shape
    blk = 4096
    grid = N // blk
    spec = pl.BlockSpec((B, blk), lambda i: (0, i))
    return pl.pallas_call(
        _idx_body,
        grid=(grid,),
        in_specs=[spec, spec, spec],
        out_specs=spec,
        out_shape=jax.ShapeDtypeStruct((B, N), jnp.int32),
    )(x, y, z)


def _sc_voxelize(features, idx):
    B, C, N = features.shape
    CPW = C // _NW
    assert CPW * _NW == C
    CHUNK = 1024
    NBUF = 4
    NCH = N // CHUNK
    assert NCH * CHUNK == N and NCH % NBUF == 0
    STEPS = CHUNK // _L
    UNROLL = 8
    STAGE = N // _NS

    mesh = plsc.VectorSubcoreMesh(
        core_axis_name="c", subcore_axis_name="s",
        num_cores=_NC, num_subcores=_NS)

    @functools.partial(
        pl.kernel,
        out_type=jax.ShapeDtypeStruct((B, C, _V), jnp.float32),
        mesh=mesh,
        compiler_params=pltpu.CompilerParams(needs_layout_passes=False),
        scratch_types=[
            pltpu.VMEM((_V,), jnp.float32),
            pltpu.VMEM((_V,), jnp.float32),
            pltpu.VMEM((_V,), jnp.float32),
            pltpu.VMEM((_V,), jnp.float32),
            pltpu.VMEM((_V,), jnp.float32),
            pltpu.VMEM((NBUF, CHUNK), jnp.int32),
            pltpu.VMEM((NBUF, 2, CHUNK), jnp.float32),
            pltpu.SemaphoreType.DMA,
            pltpu.SemaphoreType.DMA,
            pltpu.SemaphoreType.DMA,
            pltpu.SemaphoreType.DMA,
            pltpu.SemaphoreType.DMA,
            pltpu.SemaphoreType.DMA,
        ],
    )
    def body(feat_hbm, idx_hbm, out_hbm, acc0, acc1, cnt, stg0, stg1,
             idxb, fb, s0, s1, s2, s3, outsem, stgsem):
        cid = lax.axis_index("c")
        sid = lax.axis_index("s")
        wid = sid * _NC + cid
        c0 = wid * CPW

        zeros16 = jnp.zeros((_L,), jnp.float32)
        ones16 = jnp.ones((_L,), jnp.float32)
        insems = (s0, s1, s2, s3)

        def zero_all(i, carry):
            sl = pl.ds(i * _L, _L)
            acc0[sl] = zeros16
            acc1[sl] = zeros16
            cnt[sl] = zeros16
            return carry
        lax.fori_loop(0, _V // _L, zero_all, 0)

        def issue_in(b, base, buf):
            pltpu.async_copy(idx_hbm.at[b, pl.ds(base, CHUNK)],
                             idxb.at[buf], insems[buf])
            pltpu.async_copy(feat_hbm.at[b, c0, pl.ds(base, CHUNK)],
                             fb.at[buf, 0], insems[buf])
            pltpu.async_copy(feat_hbm.at[b, c0 + 1, pl.ds(base, CHUNK)],
                             fb.at[buf, 1], insems[buf])

        def wait_in(buf):
            pltpu.make_async_copy(idx_hbm.at[0, pl.ds(0, CHUNK)],
                                  idxb.at[buf], insems[buf]).wait()
            pltpu.make_async_copy(feat_hbm.at[0, 0, pl.ds(0, CHUNK)],
                                  fb.at[buf, 0], insems[buf]).wait()
            pltpu.make_async_copy(feat_hbm.at[0, 0, pl.ds(0, CHUNK)],
                                  fb.at[buf, 1], insems[buf]).wait()

        def scatter_chunk(buf):
            def step(t, carry):
                for u in range(UNROLL):
                    sl = pl.ds((t * UNROLL + u) * _L, _L)
                    iv = idxb[buf, sl]
                    f0 = fb[buf, 0, sl]
                    f1 = fb[buf, 1, sl]
                    plsc.addupdate_scatter(acc0, [iv], f0)
                    plsc.addupdate_scatter(acc1, [iv], f1)
                    plsc.addupdate_scatter(cnt, [iv], ones16)
                return carry
            lax.fori_loop(0, STEPS // UNROLL, step, 0)

        def wait_out():
            pltpu.make_async_copy(stg0, out_hbm.at[0, 0], outsem).wait()
            pltpu.make_async_copy(stg1, out_hbm.at[0, 0], outsem).wait()

        for b in range(B):
            for u in range(NBUF):
                issue_in(b, u * CHUNK, u)

            def ring(g, carry):
                base = (g + 1) * (NBUF * CHUNK)
                for u in range(NBUF):
                    wait_in(u)
                    scatter_chunk(u)
                    issue_in(b, base + u * CHUNK, u)
                return carry
            lax.fori_loop(0, NCH // NBUF - 1, ring, 0)
            for u in range(NBUF):
                wait_in(u)
                scatter_chunk(u)

            if b > 0:
                wait_out()

            def divz(i, carry):
                sl = pl.ds(i * _L, _L)
                cv = cnt[sl]
                r = 1.0 / jnp.maximum(cv, ones16)
                stg0[sl] = acc0[sl] * r
                stg1[sl] = acc1[sl] * r
                acc0[sl] = zeros16
                acc1[sl] = zeros16
                cnt[sl] = zeros16
                return carry
            lax.fori_loop(0, _V // _L, divz, 0)

            pltpu.async_copy(stg0, out_hbm.at[b, c0], outsem)
            pltpu.async_copy(stg1, out_hbm.at[b, c0 + 1], outsem)

        wait_out()

    return body(features, idx)


def kernel(features, coords):
    B, C, N = features.shape
    x = coords[:, :, 0]
    y = coords[:, :, 1]
    z = coords[:, :, 2]
    idx = _flat_idx(x, y, z)
    out = _sc_voxelize(features, idx)
    return out.reshape(B, C, _X, _Y, _Z)


# speedup vs baseline: 1.0852x; 1.0852x over previous
import functools

import jax
import jax.numpy as jnp
from jax import lax
from jax.experimental import pallas as pl
from jax.experimental.pallas import tpu as pltpu
from jax.experimental.pallas import tpu_sc as plsc

_X, _Y, _Z = 38, 24, 24
_V = _X * _Y * _Z
_VOXEL = (0.3, 0.3, 0.2)
_GROUND = (-5.6, -3.6, -2.4)
_DIMS = (_X, _Y, _Z)

_NC, _NS, _L = 2, 16, 16
_NW = _NC * _NS


def _idx_body(x_ref, y_ref, z_ref, o_ref):
    comps = []
    for ref, vs, g, dim in zip((x_ref, y_ref, z_ref), _VOXEL, _GROUND, _DIMS):
        vsf = jnp.float32(vs)
        mn = jnp.floor(jnp.float32(g) / vsf)
        d = jnp.floor(ref[...] / vsf)
        vi = (d - mn).astype(jnp.int32)
        comps.append(jnp.clip(vi, 0, dim - 1))
    o_ref[...] = comps[0] * (_Y * _Z) + comps[1] * _Z + comps[2]


def _flat_idx(x, y, z):
    B, N = x.shape
    blk = 4096
    grid = N // blk
    spec = pl.BlockSpec((B, blk), lambda i: (0, i))
    return pl.pallas_call(
        _idx_body,
        grid=(grid,),
        in_specs=[spec, spec, spec],
        out_specs=spec,
        out_shape=jax.ShapeDtypeStruct((B, N), jnp.int32),
    )(x, y, z)


def _sc_voxelize(features, idx):
    B, C, N = features.shape
    CPW = C // _NW
    assert CPW * _NW == C
    CHUNK = 1024
    NBUF = 2
    NCH = N // CHUNK
    assert NCH * CHUNK == N and NCH % NBUF == 0
    STEPS = CHUNK // _L
    UNROLL = 8
    STAGE = N // _NS

    mesh = plsc.VectorSubcoreMesh(
        core_axis_name="c", subcore_axis_name="s",
        num_cores=_NC, num_subcores=_NS)

    @functools.partial(
        pl.kernel,
        out_type=jax.ShapeDtypeStruct((B, C, _V), jnp.float32),
        mesh=mesh,
        compiler_params=pltpu.CompilerParams(needs_layout_passes=False),
        scratch_types=[
            pltpu.VMEM((_V,), jnp.float32),
            pltpu.VMEM((_V,), jnp.float32),
            pltpu.VMEM((_V,), jnp.float32),
            pltpu.VMEM((_V,), jnp.float32),
            pltpu.VMEM((_V,), jnp.float32),
            pltpu.VMEM((NBUF, CHUNK), jnp.int32),
            pltpu.VMEM((NBUF, 2, CHUNK), jnp.float32),
            pltpu.SemaphoreType.DMA,
            pltpu.SemaphoreType.DMA,
            pltpu.SemaphoreType.DMA,
        ],
    )
    def body(feat_hbm, idx_hbm, out_hbm, acc0, acc1, cnt, stg0, stg1,
             idxb, fb, s0, s1, outsem):
        cid = lax.axis_index("c")
        sid = lax.axis_index("s")
        wid = sid * _NC + cid
        c0 = wid * CPW

        zeros16 = jnp.zeros((_L,), jnp.float32)
        ones16 = jnp.ones((_L,), jnp.float32)
        insems = (s0, s1)

        @plsc.parallel_loop(0, _V // _L, 1, unroll=8)
        def zero_all(i):
            sl = pl.ds(i * _L, _L)
            acc0[sl] = zeros16
            acc1[sl] = zeros16
            cnt[sl] = zeros16

        def issue_in(b, base, buf):
            pltpu.async_copy(idx_hbm.at[b, pl.ds(base, CHUNK)],
                             idxb.at[buf], insems[buf])
            pltpu.async_copy(feat_hbm.at[b, c0, pl.ds(base, CHUNK)],
                             fb.at[buf, 0], insems[buf])
            pltpu.async_copy(feat_hbm.at[b, c0 + 1, pl.ds(base, CHUNK)],
                             fb.at[buf, 1], insems[buf])

        def wait_in(buf):
            pltpu.make_async_copy(idx_hbm.at[0, pl.ds(0, CHUNK)],
                                  idxb.at[buf], insems[buf]).wait()
            pltpu.make_async_copy(feat_hbm.at[0, 0, pl.ds(0, CHUNK)],
                                  fb.at[buf, 0], insems[buf]).wait()
            pltpu.make_async_copy(feat_hbm.at[0, 0, pl.ds(0, CHUNK)],
                                  fb.at[buf, 1], insems[buf]).wait()

        def scatter_chunk(buf):
            @plsc.parallel_loop(0, STEPS, 1, unroll=UNROLL)
            def _scatter(t):
                sl = pl.ds(t * _L, _L)
                iv = idxb[buf, sl]
                f0 = fb[buf, 0, sl]
                f1 = fb[buf, 1, sl]
                plsc.addupdate_scatter(acc0, [iv], f0)
                plsc.addupdate_scatter(acc1, [iv], f1)
                plsc.addupdate_scatter(cnt, [iv], ones16)

        def wait_out():
            pltpu.make_async_copy(stg0, out_hbm.at[0, 0], outsem).wait()
            pltpu.make_async_copy(stg1, out_hbm.at[0, 0], outsem).wait()

        def batch_body(b, carry):
            for u in range(NBUF):
                issue_in(b, u * CHUNK, u)

            def ring(g, carry2):
                base = (g + 1) * (NBUF * CHUNK)
                for u in range(NBUF):
                    wait_in(u)
                    scatter_chunk(u)
                    issue_in(b, base + u * CHUNK, u)
                return carry2
            lax.fori_loop(0, NCH // NBUF - 1, ring, 0)
            for u in range(NBUF):
                wait_in(u)
                scatter_chunk(u)

            @pl.when(b > 0)
            def _():
                wait_out()

            @plsc.parallel_loop(0, _V // _L, 1, unroll=8)
            def divz(i):
                sl = pl.ds(i * _L, _L)
                cv = cnt[sl]
                r = 1.0 / jnp.maximum(cv, ones16)
                stg0[sl] = acc0[sl] * r
                stg1[sl] = acc1[sl] * r
                acc0[sl] = zeros16
                acc1[sl] = zeros16
                cnt[sl] = zeros16

            pltpu.async_copy(stg0, out_hbm.at[b, c0], outsem)
            pltpu.async_copy(stg1, out_hbm.at[b, c0 + 1], outsem)
            return carry

        lax.fori_loop(0, B, batch_body, 0)
        wait_out()

    return body(features, idx)


def kernel(features, coords):
    B, C, N = features.shape
    x = coords[:, :, 0]
    y = coords[:, :, 1]
    z = coords[:, :, 2]
    idx = _flat_idx(x, y, z)
    out = _sc_voxelize(features, idx)
    return out.reshape(B, C, _X, _Y, _Z)


# speedup vs baseline: 1.2863x; 1.1853x over previous
import functools

import jax
import jax.numpy as jnp
from jax import lax
from jax.experimental import pallas as pl
from jax.experimental.pallas import tpu as pltpu
from jax.experimental.pallas import tpu_sc as plsc

_X, _Y, _Z = 38, 24, 24
_V = _X * _Y * _Z
_VOXEL = (0.3, 0.3, 0.2)
_GROUND = (-5.6, -3.6, -2.4)
_DIMS = (_X, _Y, _Z)

_NC, _NS, _L = 2, 16, 16
_NW = _NC * _NS


def _idx_body(x_ref, y_ref, z_ref, o_ref):
    comps = []
    for ref, vs, g, dim in zip((x_ref, y_ref, z_ref), _VOXEL, _GROUND, _DIMS):
        vsf = jnp.float32(vs)
        mn = jnp.floor(jnp.float32(g) / vsf)
        d = jnp.floor(ref[...] / vsf)
        vi = (d - mn).astype(jnp.int32)
        comps.append(jnp.clip(vi, 0, dim - 1))
    o_ref[...] = comps[0] * (_Y * _Z) + comps[1] * _Z + comps[2]


def _flat_idx(x, y, z):
    B, N = x.shape
    blk = 4096
    grid = N // blk
    spec = pl.BlockSpec((B, blk), lambda i: (0, i))
    return pl.pallas_call(
        _idx_body,
        grid=(grid,),
        in_specs=[spec, spec, spec],
        out_specs=spec,
        out_shape=jax.ShapeDtypeStruct((B, N), jnp.int32),
    )(x, y, z)


def _sc_voxelize(features, idx):
    B, C, N = features.shape
    CPW = C // _NW
    assert CPW * _NW == C
    CHUNK = 2048
    NBUF = 2
    NCH = N // CHUNK
    assert NCH * CHUNK == N and NCH % NBUF == 0
    STEPS = CHUNK // _L
    UNROLL = 16
    STAGE = N // _NS

    mesh = plsc.VectorSubcoreMesh(
        core_axis_name="c", subcore_axis_name="s",
        num_cores=_NC, num_subcores=_NS)

    @functools.partial(
        pl.kernel,
        out_type=jax.ShapeDtypeStruct((B, C, _V), jnp.float32),
        mesh=mesh,
        compiler_params=pltpu.CompilerParams(needs_layout_passes=False),
        scratch_types=[
            pltpu.VMEM((_V,), jnp.float32),
            pltpu.VMEM((_V,), jnp.float32),
            pltpu.VMEM((_V,), jnp.float32),
            pltpu.VMEM((_V,), jnp.float32),
            pltpu.VMEM((_V,), jnp.float32),
            pltpu.VMEM((NBUF, CHUNK), jnp.int32),
            pltpu.VMEM((NBUF, 2, CHUNK), jnp.float32),
            pltpu.SemaphoreType.DMA,
            pltpu.SemaphoreType.DMA,
            pltpu.SemaphoreType.DMA,
        ],
    )
    def body(feat_hbm, idx_hbm, out_hbm, acc0, acc1, cnt, stg0, stg1,
             idxb, fb, s0, s1, outsem):
        cid = lax.axis_index("c")
        sid = lax.axis_index("s")
        wid = sid * _NC + cid
        c0 = wid * CPW

        zeros16 = jnp.zeros((_L,), jnp.float32)
        ones16 = jnp.ones((_L,), jnp.float32)
        insems = (s0, s1)

        @plsc.parallel_loop(0, _V // _L, 1, unroll=8)
        def zero_all(i):
            sl = pl.ds(i * _L, _L)
            acc0[sl] = zeros16
            acc1[sl] = zeros16
            cnt[sl] = zeros16

        def issue_in(b, base, buf):
            pltpu.async_copy(idx_hbm.at[b, pl.ds(base, CHUNK)],
                             idxb.at[buf], insems[buf])
            pltpu.async_copy(feat_hbm.at[b, c0, pl.ds(base, CHUNK)],
                             fb.at[buf, 0], insems[buf])
            pltpu.async_copy(feat_hbm.at[b, c0 + 1, pl.ds(base, CHUNK)],
                             fb.at[buf, 1], insems[buf])

        def wait_in(buf):
            pltpu.make_async_copy(idx_hbm.at[0, pl.ds(0, CHUNK)],
                                  idxb.at[buf], insems[buf]).wait()
            pltpu.make_async_copy(feat_hbm.at[0, 0, pl.ds(0, CHUNK)],
                                  fb.at[buf, 0], insems[buf]).wait()
            pltpu.make_async_copy(feat_hbm.at[0, 0, pl.ds(0, CHUNK)],
                                  fb.at[buf, 1], insems[buf]).wait()

        def scatter_chunk(buf):
            @plsc.parallel_loop(0, STEPS, 1, unroll=UNROLL)
            def _scatter(t):
                sl = pl.ds(t * _L, _L)
                iv = idxb[buf, sl]
                f0 = fb[buf, 0, sl]
                f1 = fb[buf, 1, sl]
                plsc.addupdate_scatter(acc0, [iv], f0)
                plsc.addupdate_scatter(acc1, [iv], f1)
                plsc.addupdate_scatter(cnt, [iv], ones16)

        def wait_out():
            pltpu.make_async_copy(stg0, out_hbm.at[0, 0], outsem).wait()
            pltpu.make_async_copy(stg1, out_hbm.at[0, 0], outsem).wait()

        def batch_body(b, carry):
            for u in range(NBUF):
                issue_in(b, u * CHUNK, u)

            def ring(g, carry2):
                base = (g + 1) * (NBUF * CHUNK)
                for u in range(NBUF):
                    wait_in(u)
                    scatter_chunk(u)
                    issue_in(b, base + u * CHUNK, u)
                return carry2
            lax.fori_loop(0, NCH // NBUF - 1, ring, 0)
            for u in range(NBUF):
                wait_in(u)
                scatter_chunk(u)

            @pl.when(b > 0)
            def _():
                wait_out()

            @plsc.parallel_loop(0, _V // _L, 1, unroll=8)
            def divz(i):
                sl = pl.ds(i * _L, _L)
                cv = cnt[sl]
                r = 1.0 / jnp.maximum(cv, ones16)
                stg0[sl] = acc0[sl] * r
                stg1[sl] = acc1[sl] * r
                acc0[sl] = zeros16
                acc1[sl] = zeros16
                cnt[sl] = zeros16

            pltpu.async_copy(stg0, out_hbm.at[b, c0], outsem)
            pltpu.async_copy(stg1, out_hbm.at[b, c0 + 1], outsem)
            return carry

        lax.fori_loop(0, B, batch_body, 0)
        wait_out()

    return body(features, idx)


def kernel(features, coords):
    B, C, N = features.shape
    x = coords[:, :, 0]
    y = coords[:, :, 1]
    z = coords[:, :, 2]
    idx = _flat_idx(x, y, z)
    out = _sc_voxelize(features, idx)
    return out.reshape(B, C, _X, _Y, _Z)
